# baseline (speedup 1.0000x reference)
import jax
import jax.numpy as jnp
from jax.experimental import pallas as pl
from jax.experimental.pallas import tpu as pltpu

_B, _C, _H, _W = 8, 19, 512, 512
_HB = 128


_P0 = 9.975032552123064e-06
_P1 = 0.9992354838332771
_P2 = -0.4902307234234269
_P3 = 0.28527268109062165
_P4 = -0.13158182508881333
_P5 = 0.03044900453868939


def _log1p_poly(u):
    p = _P5
    for c in (_P4, _P3, _P2, _P1, _P0):
        p = p * u + c
    return p


def _body(x_ref, t_ref, out_ref):
    x = x_ref[0]
    t = t_ref[0]
    cls = jax.lax.broadcasted_iota(jnp.int32, x.shape, 0)
    a = jnp.abs(x)
    relu = (x + a) * 0.5
    u = jnp.exp(-a)
    loss = relu + _log1p_poly(u) - jnp.where(cls == t[None], x, 0.0)
    s = jnp.sum(loss)

    @pl.when((pl.program_id(0) == 0) & (pl.program_id(1) == 0))
    def _():
        out_ref[0, 0] = 0.0

    out_ref[0, 0] += s


def kernel(input, target, epoch):
    del epoch
    n = input.size
    grid = (_B, _H // _HB)
    out = pl.pallas_call(
        _body,
        grid=grid,
        in_specs=[
            pl.BlockSpec((1, _C, _HB, _W), lambda b, h: (b, 0, h, 0)),
            pl.BlockSpec((1, _HB, _W), lambda b, h: (b, h, 0)),
        ],
        out_specs=pl.BlockSpec(memory_space=pltpu.SMEM),
        out_shape=jax.ShapeDtypeStruct((1, 1), jnp.float32),
    )(input, target)
    return out[0, 0] / n


# speedup vs baseline: 2.7052x; 2.7052x over previous
import jax
import jax.numpy as jnp
from jax.experimental import pallas as pl
from jax.experimental.pallas import tpu as pltpu

_B, _C, _H, _W = 8, 19, 512, 512
_HB = 128


_NEG_LOG2E = -1.4426950408889634
_LN2 = 0.6931471805599453


def _body(x_ref, t_ref, out_ref):
    def chunk(i, carry):
        acc1, acc2 = carry
        r = i * 8
        t = t_ref[0, pl.ds(r, 8), :]
        for c in range(_C):
            xc = x_ref[0, c, pl.ds(r, 8), :]
            a = jnp.abs(xc)
            zpa = jnp.where(t == c, a - xc, a + xc)
            u = jnp.exp2(a * _NEG_LOG2E)
            l = jnp.log2(1.0 + u)
            acc1 = acc1 + zpa
            acc2 = acc2 + l
        return acc1, acc2

    z = jnp.zeros((8, _W), jnp.float32)
    acc1, acc2 = jax.lax.fori_loop(0, _HB // 8, chunk, (z, z))
    s = 0.5 * jnp.sum(acc1) + _LN2 * jnp.sum(acc2)

    @pl.when((pl.program_id(0) == 0) & (pl.program_id(1) == 0))
    def _():
        out_ref[0, 0] = 0.0

    out_ref[0, 0] += s


def kernel(input, target, epoch):
    del epoch
    n = input.size
    grid = (_B, _H // _HB)
    out = pl.pallas_call(
        _body,
        grid=grid,
        in_specs=[
            pl.BlockSpec((1, _C, _HB, _W), lambda b, h: (b, 0, h, 0)),
            pl.BlockSpec((1, _HB, _W), lambda b, h: (b, h, 0)),
        ],
        out_specs=pl.BlockSpec(memory_space=pltpu.SMEM),
        out_shape=jax.ShapeDtypeStruct((1, 1), jnp.float32),
    )(input, target)
    return out[0, 0] / n
